# baseline (speedup 1.0000x reference)
import jax
import jax.numpy as jnp
from jax import lax
from jax.experimental import pallas as pl
from jax.experimental.pallas import tpu as pltpu
from jax.experimental.pallas import tpu_sc as plsc

N = 8388608
NBINS = 1024
A = 0.05
B = 0.95
XI = 0.1

NC = 2
NS = 16
L = 16
NW = NC * NS
PER_TILE = N // NW
CHUNK = 16384
NCHUNK = PER_TILE // CHUNK
TAB = 1040
UNROLL = 1


def _body(x_hbm, edges_hbm, cdf_hbm, par_hbm, out_hbm,
          edges_v, cdf_v, par_v, p_v, q_v, r_v,
          xb0, xb1, ub0, ub1, si0, si1, so0, so1):
    xb = (xb0, xb1)
    ubuf = (ub0, ub1)
    si = (si0, si1)
    so = (so0, so1)
    wid = lax.axis_index("s") * NC + lax.axis_index("c")
    base = wid * PER_TILE

    in_desc = [None] * NCHUNK
    out_desc = [None] * NCHUNK
    in_desc[0] = pltpu.async_copy(
        x_hbm.at[pl.ds(base, CHUNK)], xb[0], si[0])
    in_desc[1] = pltpu.async_copy(
        x_hbm.at[pl.ds(base + CHUNK, CHUNK)], xb[1], si[1])

    pltpu.sync_copy(edges_hbm, edges_v)
    pltpu.sync_copy(cdf_hbm, cdf_v)
    pltpu.sync_copy(par_hbm, par_v)

    c_mul = par_v[pl.ds(0, L)]
    c_add = par_v[pl.ds(L, L)]
    c_mid = par_v[pl.ds(2 * L, L)]
    c_base = par_v[pl.ds(3 * L, L)]
    inv_heps = par_v[pl.ds(4 * L, L)]

    iota = lax.iota(jnp.int32, L)

    @pl.loop(0, TAB, step=L)
    def _tab(i):
        pos = iota + i
        pm = jnp.minimum(jnp.maximum(pos - 1, 0), NBINS)
        pc = jnp.minimum(pos, NBINS)
        y0 = plsc.load_gather(cdf_v, [pm])
        y1 = plsc.load_gather(cdf_v, [pc])
        e0 = plsc.load_gather(edges_v, [pm])
        sl = (y1 - y0) * inv_heps
        is_lo = pos == 0
        is_hi = pos == NBINS + 1
        p_v[pl.ds(i, L)] = jnp.where(
            is_lo, 0.0, jnp.where(is_hi, 1.0, y0 - sl * e0))
        q_v[pl.ds(i, L)] = sl
        r_v[pl.ds(i, L)] = jnp.where(
            is_lo, A, jnp.where(is_hi, B - 1.0, 0.0))

    for c in range(NCHUNK):
        b = c & 1
        in_desc[c].wait()
        if c >= 2:
            out_desc[c - 2].wait()

        @plsc.parallel_loop(0, CHUNK, step=L, unroll=UNROLL)
        def _vec(i, _b=b):
            xv = xb[_b][pl.ds(i, L)]
            t = xv * c_mul + c_add
            t = jnp.minimum(jnp.maximum(t, 0.0), float(NBINS + 1))
            j = t.astype(jnp.int32)
            pj = plsc.load_gather(p_v, [j])
            qj = plsc.load_gather(q_v, [j])
            rj = plsc.load_gather(r_v, [j])
            b1 = jnp.abs(xv - c_mid) * XI + c_base
            b2 = b1 * b1
            b4 = b2 * b2
            b8 = b4 * b4
            s = 1.0 / (b8 * b2)
            ubuf[_b][pl.ds(i, L)] = pj + qj * xv + rj * s

        out_desc[c] = pltpu.async_copy(
            ubuf[b], out_hbm.at[pl.ds(base + c * CHUNK, CHUNK)], so[b])
        if c + 2 < NCHUNK:
            in_desc[c + 2] = pltpu.async_copy(
                x_hbm.at[pl.ds(base + (c + 2) * CHUNK, CHUNK)], xb[b], si[b])

    out_desc[NCHUNK - 2].wait()
    out_desc[NCHUNK - 1].wait()


@jax.jit
def _marginal_sc(x, edges_pad, cdf_pad, params):
    fn = pl.kernel(
        _body,
        out_type=jax.ShapeDtypeStruct((N,), jnp.float32),
        mesh=plsc.VectorSubcoreMesh(
            core_axis_name="c", subcore_axis_name="s",
            num_cores=NC, num_subcores=NS),
        scratch_types=[
            pltpu.VMEM((TAB,), jnp.float32),
            pltpu.VMEM((TAB,), jnp.float32),
            pltpu.VMEM((5 * L,), jnp.float32),
            pltpu.VMEM((TAB,), jnp.float32),
            pltpu.VMEM((TAB,), jnp.float32),
            pltpu.VMEM((TAB,), jnp.float32),
            pltpu.VMEM((CHUNK,), jnp.float32),
            pltpu.VMEM((CHUNK,), jnp.float32),
            pltpu.VMEM((CHUNK,), jnp.float32),
            pltpu.VMEM((CHUNK,), jnp.float32),
            pltpu.SemaphoreType.DMA,
            pltpu.SemaphoreType.DMA,
            pltpu.SemaphoreType.DMA,
            pltpu.SemaphoreType.DMA,
        ],
        compiler_params=pltpu.CompilerParams(needs_layout_passes=False),
    )
    return fn(x, edges_pad, cdf_pad, params)


def kernel(x, bin_edges, cdf_vals):
    alpha = bin_edges[0]
    beta = bin_edges[-1]
    h = (beta - alpha) / NBINS
    eps = jnp.finfo(jnp.float32).eps
    params = jnp.concatenate([
        jnp.full((L,), 1.0 / h, jnp.float32),
        jnp.full((L,), 1.0 - alpha / h, jnp.float32),
        jnp.full((L,), (alpha + beta) / 2.0, jnp.float32),
        jnp.full((L,), 1.0 - XI * (beta - alpha) / 2.0, jnp.float32),
        jnp.full((L,), 1.0 / (eps + h), jnp.float32),
    ])
    edges_pad = jnp.concatenate(
        [bin_edges, jnp.zeros((TAB - NBINS - 1,), jnp.float32)])
    cdf_pad = jnp.concatenate(
        [cdf_vals, jnp.zeros((TAB - NBINS - 1,), jnp.float32)])
    return _marginal_sc(x, edges_pad, cdf_pad, params)


# speedup vs baseline: 1.1404x; 1.1404x over previous
import jax
import jax.numpy as jnp
from jax import lax
from jax.experimental import pallas as pl
from jax.experimental.pallas import tpu as pltpu
from jax.experimental.pallas import tpu_sc as plsc

N = 8388608
NBINS = 1024
A = 0.05
B = 0.95
XI = 0.1

NC = 2
NS = 16
L = 16
NW = NC * NS
PER_TILE = N // NW
CHUNK = 8192
NCHUNK = PER_TILE // CHUNK
TAB = 1040
UNROLL = 2


def _body(x_hbm, edges_hbm, cdf_hbm, par_hbm, out_hbm,
          edges_v, cdf_v, par_v, p_v, q_v, r_v,
          xb0, xb1, ub0, ub1, si0, si1, so0, so1):
    xb = (xb0, xb1)
    ubuf = (ub0, ub1)
    si = (si0, si1)
    so = (so0, so1)
    wid = lax.axis_index("s") * NC + lax.axis_index("c")
    base = wid * PER_TILE

    in_desc = [None] * NCHUNK
    out_desc = [None] * NCHUNK
    in_desc[0] = pltpu.async_copy(
        x_hbm.at[pl.ds(base, CHUNK)], xb[0], si[0])
    in_desc[1] = pltpu.async_copy(
        x_hbm.at[pl.ds(base + CHUNK, CHUNK)], xb[1], si[1])

    pltpu.sync_copy(edges_hbm, edges_v)
    pltpu.sync_copy(cdf_hbm, cdf_v)
    pltpu.sync_copy(par_hbm, par_v)

    c_mul = par_v[pl.ds(0, L)]
    c_add = par_v[pl.ds(L, L)]
    c_mid = par_v[pl.ds(2 * L, L)]
    c_base = par_v[pl.ds(3 * L, L)]
    inv_heps = par_v[pl.ds(4 * L, L)]

    iota = lax.iota(jnp.int32, L)

    @pl.loop(0, TAB, step=L)
    def _tab(i):
        pos = iota + i
        pm = jnp.minimum(jnp.maximum(pos - 1, 0), NBINS)
        pc = jnp.minimum(pos, NBINS)
        y0 = plsc.load_gather(cdf_v, [pm])
        y1 = plsc.load_gather(cdf_v, [pc])
        e0 = plsc.load_gather(edges_v, [pm])
        sl = (y1 - y0) * inv_heps
        is_lo = pos == 0
        is_hi = pos == NBINS + 1
        p_v[pl.ds(i, L)] = jnp.where(
            is_lo, 0.0, jnp.where(is_hi, 1.0, y0 - sl * e0))
        q_v[pl.ds(i, L)] = sl
        r_v[pl.ds(i, L)] = jnp.where(
            is_lo, A, jnp.where(is_hi, B - 1.0, 0.0))

    for c in range(NCHUNK):
        b = c & 1
        in_desc[c].wait()
        if c >= 2:
            out_desc[c - 2].wait()

        @plsc.parallel_loop(0, CHUNK, step=L, unroll=UNROLL)
        def _vec(i, _b=b):
            xv = xb[_b][pl.ds(i, L)]
            t = xv * c_mul + c_add
            t = jnp.minimum(jnp.maximum(t, 0.0), float(NBINS + 1))
            j = t.astype(jnp.int32)
            pj = plsc.load_gather(p_v, [j])
            qj = plsc.load_gather(q_v, [j])
            rj = plsc.load_gather(r_v, [j])
            b1 = jnp.abs(xv - c_mid) * XI + c_base
            b2 = b1 * b1
            b4 = b2 * b2
            b8 = b4 * b4
            s = 1.0 / (b8 * b2)
            ubuf[_b][pl.ds(i, L)] = pj + qj * xv + rj * s

        out_desc[c] = pltpu.async_copy(
            ubuf[b], out_hbm.at[pl.ds(base + c * CHUNK, CHUNK)], so[b])
        if c + 2 < NCHUNK:
            in_desc[c + 2] = pltpu.async_copy(
                x_hbm.at[pl.ds(base + (c + 2) * CHUNK, CHUNK)], xb[b], si[b])

    out_desc[NCHUNK - 2].wait()
    out_desc[NCHUNK - 1].wait()


@jax.jit
def _marginal_sc(x, edges_pad, cdf_pad, params):
    fn = pl.kernel(
        _body,
        out_type=jax.ShapeDtypeStruct((N,), jnp.float32),
        mesh=plsc.VectorSubcoreMesh(
            core_axis_name="c", subcore_axis_name="s",
            num_cores=NC, num_subcores=NS),
        scratch_types=[
            pltpu.VMEM((TAB,), jnp.float32),
            pltpu.VMEM((TAB,), jnp.float32),
            pltpu.VMEM((5 * L,), jnp.float32),
            pltpu.VMEM((TAB,), jnp.float32),
            pltpu.VMEM((TAB,), jnp.float32),
            pltpu.VMEM((TAB,), jnp.float32),
            pltpu.VMEM((CHUNK,), jnp.float32),
            pltpu.VMEM((CHUNK,), jnp.float32),
            pltpu.VMEM((CHUNK,), jnp.float32),
            pltpu.VMEM((CHUNK,), jnp.float32),
            pltpu.SemaphoreType.DMA,
            pltpu.SemaphoreType.DMA,
            pltpu.SemaphoreType.DMA,
            pltpu.SemaphoreType.DMA,
        ],
        compiler_params=pltpu.CompilerParams(needs_layout_passes=False),
    )
    return fn(x, edges_pad, cdf_pad, params)


def kernel(x, bin_edges, cdf_vals):
    alpha = bin_edges[0]
    beta = bin_edges[-1]
    h = (beta - alpha) / NBINS
    eps = jnp.finfo(jnp.float32).eps
    params = jnp.concatenate([
        jnp.full((L,), 1.0 / h, jnp.float32),
        jnp.full((L,), 1.0 - alpha / h, jnp.float32),
        jnp.full((L,), (alpha + beta) / 2.0, jnp.float32),
        jnp.full((L,), 1.0 - XI * (beta - alpha) / 2.0, jnp.float32),
        jnp.full((L,), 1.0 / (eps + h), jnp.float32),
    ])
    edges_pad = jnp.concatenate(
        [bin_edges, jnp.zeros((TAB - NBINS - 1,), jnp.float32)])
    cdf_pad = jnp.concatenate(
        [cdf_vals, jnp.zeros((TAB - NBINS - 1,), jnp.float32)])
    return _marginal_sc(x, edges_pad, cdf_pad, params)


# speedup vs baseline: 1.1499x; 1.0083x over previous
import jax
import jax.numpy as jnp
from jax import lax
from jax.experimental import pallas as pl
from jax.experimental.pallas import tpu as pltpu
from jax.experimental.pallas import tpu_sc as plsc

N = 8388608
NBINS = 1024
A = 0.05
B = 0.95
XI = 0.1

NC = 2
NS = 16
L = 16
NW = NC * NS
PER_TILE = N // NW
CHUNK = 16384
NCHUNK = PER_TILE // CHUNK
TAB = 1040
UNROLL = 2


def _body(x_hbm, edges_hbm, cdf_hbm, par_hbm, out_hbm,
          edges_v, cdf_v, par_v, p_v, q_v, r_v,
          xb0, xb1, ub0, ub1, si0, si1, so0, so1):
    xb = (xb0, xb1)
    ubuf = (ub0, ub1)
    si = (si0, si1)
    so = (so0, so1)
    wid = lax.axis_index("s") * NC + lax.axis_index("c")
    base = wid * PER_TILE

    in_desc = [None] * NCHUNK
    out_desc = [None] * NCHUNK
    in_desc[0] = pltpu.async_copy(
        x_hbm.at[pl.ds(base, CHUNK)], xb[0], si[0])
    in_desc[1] = pltpu.async_copy(
        x_hbm.at[pl.ds(base + CHUNK, CHUNK)], xb[1], si[1])

    pltpu.sync_copy(edges_hbm, edges_v)
    pltpu.sync_copy(cdf_hbm, cdf_v)
    pltpu.sync_copy(par_hbm, par_v)

    c_mul = par_v[pl.ds(0, L)]
    c_add = par_v[pl.ds(L, L)]
    c_mid = par_v[pl.ds(2 * L, L)]
    c_base = par_v[pl.ds(3 * L, L)]
    inv_heps = par_v[pl.ds(4 * L, L)]

    iota = lax.iota(jnp.int32, L)

    @pl.loop(0, TAB, step=L)
    def _tab(i):
        pos = iota + i
        pm = jnp.minimum(jnp.maximum(pos - 1, 0), NBINS)
        pc = jnp.minimum(pos, NBINS)
        y0 = plsc.load_gather(cdf_v, [pm])
        y1 = plsc.load_gather(cdf_v, [pc])
        e0 = plsc.load_gather(edges_v, [pm])
        sl = (y1 - y0) * inv_heps
        is_lo = pos == 0
        is_hi = pos == NBINS + 1
        p_v[pl.ds(i, L)] = jnp.where(
            is_lo, 0.0, jnp.where(is_hi, 1.0, y0 - sl * e0))
        q_v[pl.ds(i, L)] = sl
        r_v[pl.ds(i, L)] = jnp.where(
            is_lo, A, jnp.where(is_hi, B - 1.0, 0.0))

    for c in range(NCHUNK):
        b = c & 1
        in_desc[c].wait()
        if c >= 2:
            out_desc[c - 2].wait()

        @plsc.parallel_loop(0, CHUNK, step=L, unroll=UNROLL)
        def _vec(i, _b=b):
            xv = xb[_b][pl.ds(i, L)]
            t = xv * c_mul + c_add
            t = jnp.minimum(jnp.maximum(t, 0.0), float(NBINS + 1))
            j = t.astype(jnp.int32)
            pj = plsc.load_gather(p_v, [j])
            qj = plsc.load_gather(q_v, [j])
            rj = plsc.load_gather(r_v, [j])
            b1 = jnp.abs(xv - c_mid) * XI + c_base
            b2 = b1 * b1
            b4 = b2 * b2
            b8 = b4 * b4
            s = 1.0 / (b8 * b2)
            ubuf[_b][pl.ds(i, L)] = pj + qj * xv + rj * s

        out_desc[c] = pltpu.async_copy(
            ubuf[b], out_hbm.at[pl.ds(base + c * CHUNK, CHUNK)], so[b])
        if c + 2 < NCHUNK:
            in_desc[c + 2] = pltpu.async_copy(
                x_hbm.at[pl.ds(base + (c + 2) * CHUNK, CHUNK)], xb[b], si[b])

    out_desc[NCHUNK - 2].wait()
    out_desc[NCHUNK - 1].wait()


@jax.jit
def _marginal_sc(x, edges_pad, cdf_pad, params):
    fn = pl.kernel(
        _body,
        out_type=jax.ShapeDtypeStruct((N,), jnp.float32),
        mesh=plsc.VectorSubcoreMesh(
            core_axis_name="c", subcore_axis_name="s",
            num_cores=NC, num_subcores=NS),
        scratch_types=[
            pltpu.VMEM((TAB,), jnp.float32),
            pltpu.VMEM((TAB,), jnp.float32),
            pltpu.VMEM((5 * L,), jnp.float32),
            pltpu.VMEM((TAB,), jnp.float32),
            pltpu.VMEM((TAB,), jnp.float32),
            pltpu.VMEM((TAB,), jnp.float32),
            pltpu.VMEM((CHUNK,), jnp.float32),
            pltpu.VMEM((CHUNK,), jnp.float32),
            pltpu.VMEM((CHUNK,), jnp.float32),
            pltpu.VMEM((CHUNK,), jnp.float32),
            pltpu.SemaphoreType.DMA,
            pltpu.SemaphoreType.DMA,
            pltpu.SemaphoreType.DMA,
            pltpu.SemaphoreType.DMA,
        ],
        compiler_params=pltpu.CompilerParams(needs_layout_passes=False),
    )
    return fn(x, edges_pad, cdf_pad, params)


def kernel(x, bin_edges, cdf_vals):
    alpha = bin_edges[0]
    beta = bin_edges[-1]
    h = (beta - alpha) / NBINS
    eps = jnp.finfo(jnp.float32).eps
    params = jnp.concatenate([
        jnp.full((L,), 1.0 / h, jnp.float32),
        jnp.full((L,), 1.0 - alpha / h, jnp.float32),
        jnp.full((L,), (alpha + beta) / 2.0, jnp.float32),
        jnp.full((L,), 1.0 - XI * (beta - alpha) / 2.0, jnp.float32),
        jnp.full((L,), 1.0 / (eps + h), jnp.float32),
    ])
    edges_pad = jnp.concatenate(
        [bin_edges, jnp.zeros((TAB - NBINS - 1,), jnp.float32)])
    cdf_pad = jnp.concatenate(
        [cdf_vals, jnp.zeros((TAB - NBINS - 1,), jnp.float32)])
    return _marginal_sc(x, edges_pad, cdf_pad, params)


# speedup vs baseline: 1.1850x; 1.0305x over previous
import jax
import jax.numpy as jnp
from jax import lax
from jax.experimental import pallas as pl
from jax.experimental.pallas import tpu as pltpu
from jax.experimental.pallas import tpu_sc as plsc

N = 8388608
NBINS = 1024
A = 0.05
B = 0.95
XI = 0.1

NC = 2
NS = 16
L = 16
NW = NC * NS
PER_TILE = N // NW
CHUNK = 16384
NCHUNK = PER_TILE // CHUNK
TAB = 1040
NPAR = 7
STAGE = TAB + NPAR * L
UNROLL = 2


def _body(x_hbm, stage_hbm, out_hbm,
          stage_v, p_v, q_v, r_v,
          xb0, xb1, ub0, ub1, si0, si1, so0, so1):
    xb = (xb0, xb1)
    ubuf = (ub0, ub1)
    si = (si0, si1)
    so = (so0, so1)
    wid = lax.axis_index("s") * NC + lax.axis_index("c")
    base = wid * PER_TILE

    in_desc = [None] * NCHUNK
    out_desc = [None] * NCHUNK
    in_desc[0] = pltpu.async_copy(
        x_hbm.at[pl.ds(base, CHUNK)], xb[0], si[0])
    in_desc[1] = pltpu.async_copy(
        x_hbm.at[pl.ds(base + CHUNK, CHUNK)], xb[1], si[1])

    pltpu.sync_copy(stage_hbm, stage_v)

    c_mul = stage_v[pl.ds(TAB, L)]
    c_add = stage_v[pl.ds(TAB + L, L)]
    c_mid = stage_v[pl.ds(TAB + 2 * L, L)]
    c_base = stage_v[pl.ds(TAB + 3 * L, L)]
    inv_heps = stage_v[pl.ds(TAB + 4 * L, L)]
    alpha_v = stage_v[pl.ds(TAB + 5 * L, L)]
    h_v = stage_v[pl.ds(TAB + 6 * L, L)]

    iota = lax.iota(jnp.int32, L)

    @pl.loop(0, TAB, step=L)
    def _tab(i):
        pos = iota + i
        pm = jnp.minimum(jnp.maximum(pos - 1, 0), NBINS)
        pc = jnp.minimum(pos, NBINS)
        y0 = plsc.load_gather(stage_v, [pm])
        y1 = plsc.load_gather(stage_v, [pc])
        e0 = pm.astype(jnp.float32) * h_v + alpha_v
        sl = (y1 - y0) * inv_heps
        is_lo = pos == 0
        is_hi = pos == NBINS + 1
        p_v[pl.ds(i, L)] = jnp.where(
            is_lo, 0.0, jnp.where(is_hi, 1.0, y0 - sl * e0))
        q_v[pl.ds(i, L)] = sl
        r_v[pl.ds(i, L)] = jnp.where(
            is_lo, A, jnp.where(is_hi, B - 1.0, 0.0))

    for c in range(NCHUNK):
        b = c & 1
        in_desc[c].wait()
        if c >= 2:
            out_desc[c - 2].wait()

        @plsc.parallel_loop(0, CHUNK, step=L, unroll=UNROLL)
        def _vec(i, _b=b):
            xv = xb[_b][pl.ds(i, L)]
            t = xv * c_mul + c_add
            t = jnp.minimum(jnp.maximum(t, 0.0), float(NBINS + 1))
            j = t.astype(jnp.int32)
            pj = plsc.load_gather(p_v, [j])
            qj = plsc.load_gather(q_v, [j])
            rj = plsc.load_gather(r_v, [j])
            b1 = jnp.abs(xv - c_mid) * XI + c_base
            b2 = b1 * b1
            b4 = b2 * b2
            b8 = b4 * b4
            s = 1.0 / (b8 * b2)
            ubuf[_b][pl.ds(i, L)] = pj + qj * xv + rj * s

        out_desc[c] = pltpu.async_copy(
            ubuf[b], out_hbm.at[pl.ds(base + c * CHUNK, CHUNK)], so[b])
        if c + 2 < NCHUNK:
            in_desc[c + 2] = pltpu.async_copy(
                x_hbm.at[pl.ds(base + (c + 2) * CHUNK, CHUNK)], xb[b], si[b])

    out_desc[NCHUNK - 2].wait()
    out_desc[NCHUNK - 1].wait()


@jax.jit
def _marginal_sc(x, stage):
    fn = pl.kernel(
        _body,
        out_type=jax.ShapeDtypeStruct((N,), jnp.float32),
        mesh=plsc.VectorSubcoreMesh(
            core_axis_name="c", subcore_axis_name="s",
            num_cores=NC, num_subcores=NS),
        scratch_types=[
            pltpu.VMEM((STAGE,), jnp.float32),
            pltpu.VMEM((TAB,), jnp.float32),
            pltpu.VMEM((TAB,), jnp.float32),
            pltpu.VMEM((TAB,), jnp.float32),
            pltpu.VMEM((CHUNK,), jnp.float32),
            pltpu.VMEM((CHUNK,), jnp.float32),
            pltpu.VMEM((CHUNK,), jnp.float32),
            pltpu.VMEM((CHUNK,), jnp.float32),
            pltpu.SemaphoreType.DMA,
            pltpu.SemaphoreType.DMA,
            pltpu.SemaphoreType.DMA,
            pltpu.SemaphoreType.DMA,
        ],
        compiler_params=pltpu.CompilerParams(needs_layout_passes=False),
    )
    return fn(x, stage)


def kernel(x, bin_edges, cdf_vals):
    alpha = bin_edges[0]
    beta = bin_edges[-1]
    h = (beta - alpha) / NBINS
    eps = jnp.finfo(jnp.float32).eps
    stage = jnp.concatenate([
        cdf_vals,
        jnp.zeros((TAB - NBINS - 1,), jnp.float32),
        jnp.full((L,), 1.0 / h, jnp.float32),
        jnp.full((L,), 1.0 - alpha / h, jnp.float32),
        jnp.full((L,), (alpha + beta) / 2.0, jnp.float32),
        jnp.full((L,), 1.0 - XI * (beta - alpha) / 2.0, jnp.float32),
        jnp.full((L,), 1.0 / (eps + h), jnp.float32),
        jnp.full((L,), alpha, jnp.float32),
        jnp.full((L,), h, jnp.float32),
    ])
    return _marginal_sc(x, stage)
